# Initial kernel scaffold; baseline (speedup 1.0000x reference)
import jax
import jax.numpy as jnp
from jax.experimental import pallas as pl


def kernel(feature, adj_self, adj_dele, weight):
    raise NotImplementedError("write your pallas kernel here")


# speedup vs baseline: 1.9174x; 1.9174x over previous
import functools

import jax
import jax.numpy as jnp
from jax.experimental import pallas as pl


N = 4096
BM = 256


def _mm_kernel(a_ref, x_ref, o_ref):
    o_ref[...] = jnp.dot(a_ref[...], x_ref[...],
                         preferred_element_type=jnp.float32)


def _mm_w_kernel(a_ref, x_ref, w_ref, o_ref):
    t = jnp.dot(a_ref[...], x_ref[...], preferred_element_type=jnp.float32)
    o_ref[...] = jnp.dot(t, w_ref[...], preferred_element_type=jnp.float32)


@functools.partial(jax.jit, static_argnames=())
def _mm(mat, x):
    n, f = x.shape
    return pl.pallas_call(
        _mm_kernel,
        grid=(n // BM,),
        in_specs=[
            pl.BlockSpec((BM, n), lambda i: (i, 0)),
            pl.BlockSpec((n, f), lambda i: (0, 0)),
        ],
        out_specs=pl.BlockSpec((BM, f), lambda i: (i, 0)),
        out_shape=jax.ShapeDtypeStruct((n, f), jnp.float32),
    )(mat, x)


@functools.partial(jax.jit, static_argnames=())
def _mm_w(mat, x, w):
    n, f = x.shape
    emb = w.shape[1]
    return pl.pallas_call(
        _mm_w_kernel,
        grid=(n // BM,),
        in_specs=[
            pl.BlockSpec((BM, n), lambda i: (i, 0)),
            pl.BlockSpec((n, f), lambda i: (0, 0)),
            pl.BlockSpec((f, emb), lambda i: (0, 0)),
        ],
        out_specs=pl.BlockSpec((BM, emb), lambda i: (i, 0)),
        out_shape=jax.ShapeDtypeStruct((n, emb), jnp.float32),
    )(mat, x, w)


def kernel(feature, adj_self, adj_dele, weight):
    t = _mm(adj_dele, feature)
    t = _mm(adj_self, t)
    t = _mm(adj_dele, t)
    return _mm_w(adj_self, t, weight)
